# Initial kernel scaffold; baseline (speedup 1.0000x reference)
import jax
import jax.numpy as jnp
from jax.experimental import pallas as pl


def kernel(emg_x, emg_edge_index, emg_edge_attr, eeg_x, eeg_edge_index, eeg_edge_attr, params):
    raise NotImplementedError("write your pallas kernel here")


# speedup vs baseline: 52.3252x; 52.3252x over previous
import functools

import jax
import jax.numpy as jnp
from jax import lax
from jax.experimental import pallas as pl
from jax.experimental.pallas import tpu as pltpu
from jax.experimental.pallas import tpu_sc as plsc

_N = 256
_E = 4096
_NC = 2
_NS = 16
_EPT = _E // _NS
_LANES = 16


def _pre_body(x, w1, av1, ea, we1, ae1, we2, ae2, h1_o, sd1_o, esc_o):
    h = jnp.dot(x[...], w1[...], preferred_element_type=jnp.float32)
    h1_o[...] = h
    sd1_o[...] = jnp.dot(h, av1[...], preferred_element_type=jnp.float32)
    wc1 = jnp.sum(we1[...] * ae1[...], axis=1)
    wc2 = jnp.sum(we2[...] * ae2[...], axis=1)
    e1 = jnp.sum(ea[...] * wc1[None, :], axis=1)
    e2 = jnp.sum(ea[...] * wc2[None, :], axis=1)
    esc_o[...] = jnp.stack([e1, e2], axis=1)


def _tc_pre(x, w1, av1, ea, we1, ae1, we2, ae2):
    return pl.pallas_call(
        _pre_body,
        out_shape=[
            jax.ShapeDtypeStruct((_N, w1.shape[1]), jnp.float32),
            jax.ShapeDtypeStruct((_N, 2), jnp.float32),
            jax.ShapeDtypeStruct((_E, 2), jnp.float32),
        ],
    )(x, w1, av1, ea, we1, ae1, we2, ae2)


def _sc_edge(ssrc, sdst, esc, src, dst, zeros):
    mesh = plsc.VectorSubcoreMesh(
        core_axis_name="c", subcore_axis_name="s",
        num_cores=_NC, num_subcores=_NS)

    @functools.partial(
        pl.kernel,
        out_type=jax.ShapeDtypeStruct((2, _N * _N), jnp.float32),
        mesh=mesh,
        scratch_types=[
            pltpu.VMEM((_N,), jnp.float32),
            pltpu.VMEM((_N,), jnp.float32),
            pltpu.VMEM((_EPT,), jnp.float32),
            pltpu.VMEM((_EPT,), jnp.int32),
            pltpu.VMEM((_EPT,), jnp.int32),
            pltpu.VMEM((2, 128), jnp.float32),
            pltpu.VMEM((2, 128), jnp.int32),
            pltpu.VMEM_SHARED((_N * _N,), jnp.float32),
        ],
        compiler_params=pltpu.CompilerParams(needs_layout_passes=False),
    )
    def k(ssrc_h, sdst_h, esc_h, src_h, dst_h, zeros_h, a_out,
          ssrc_v, sdst_v, esc_v, src_v, dst_v, ex_v, idx_v, a_sh):
        c = lax.axis_index("c")
        s = lax.axis_index("s")
        base = s * _EPT
        pltpu.sync_copy(ssrc_h.at[c], ssrc_v)
        pltpu.sync_copy(sdst_h.at[c], sdst_v)
        pltpu.sync_copy(esc_h.at[c, pl.ds(base, _EPT)], esc_v)
        pltpu.sync_copy(src_h.at[c, pl.ds(base, _EPT)], src_v)
        pltpu.sync_copy(dst_h.at[c, pl.ds(base, _EPT)], dst_v)

        @pl.when(s == 0)
        def _():
            pltpu.sync_copy(zeros_h, a_sh)

        for j in range(_EPT // _LANES):
            si = src_v[pl.ds(j * _LANES, _LANES)]
            di = dst_v[pl.ds(j * _LANES, _LANES)]
            sa = plsc.load_gather(ssrc_v, [si])
            sb = plsc.load_gather(sdst_v, [di])
            t = sa + sb + esc_v[pl.ds(j * _LANES, _LANES)]
            t = jnp.maximum(t, t * jnp.float32(0.2))
            ex_v[j // 8, pl.ds((j % 8) * _LANES, _LANES)] = jnp.exp(t)
            idx_v[j // 8, pl.ds((j % 8) * _LANES, _LANES)] = di * _N + si

        plsc.subcore_barrier()
        for g in range(2):
            pltpu.sync_copy(ex_v.at[g], a_sh.at[idx_v.at[g]], add=True)
        plsc.subcore_barrier()

        @pl.when(s == 0)
        def _():
            pltpu.sync_copy(a_sh, a_out.at[c])

    return k(ssrc, sdst, esc, src, dst, zeros)


def _mid_body(a, h1, b1, w2, av2, h2_o, sd2_o):
    am = a[...]
    den = jnp.sum(am, axis=1, keepdims=True) + jnp.float32(1e-16)
    m = jnp.dot(am, h1[...], preferred_element_type=jnp.float32) / den
    t = jnp.maximum(m + b1[...], 0.0)
    h2 = jnp.dot(t, w2[...], preferred_element_type=jnp.float32)
    h2_o[...] = h2
    sd2_o[...] = jnp.dot(h2, av2[...], preferred_element_type=jnp.float32)


def _tc_mid(a, h1, b1, w2, av2):
    return pl.pallas_call(
        _mid_body,
        out_shape=[
            jax.ShapeDtypeStruct((_N, w2.shape[1]), jnp.float32),
            jax.ShapeDtypeStruct((_N, 2), jnp.float32),
        ],
    )(a, h1, b1, w2, av2)


def _fin_body(a_e, h_e, b_e, wp_e, bp_e, a_g, h_g, b_g, wp_g, bp_g,
              w1a, b1a, w1b, b1b, w2a, b2a, w2b, b2b, out_o):
    feats = []
    for a, h, b, wp, bp in ((a_e, h_e, b_e, wp_e, bp_e),
                            (a_g, h_g, b_g, wp_g, bp_g)):
        am = a[...]
        den = jnp.sum(am, axis=1, keepdims=True) + jnp.float32(1e-16)
        o = jnp.dot(am, h[...], preferred_element_type=jnp.float32) / den
        o = o + b[...]
        feats.append(jnp.dot(o, wp[...], preferred_element_type=jnp.float32)
                     + bp[...])
    z = jnp.concatenate(feats, axis=0)
    t = z + jnp.sum(z, axis=0, keepdims=True)
    t = jnp.maximum(jnp.dot(t, w1a[...], preferred_element_type=jnp.float32)
                    + b1a[...], 0.0)
    t = jnp.dot(t, w1b[...], preferred_element_type=jnp.float32) + b1b[...]
    t = jnp.maximum(t, 0.0)
    t = t + jnp.sum(t, axis=0, keepdims=True)
    t = jnp.maximum(jnp.dot(t, w2a[...], preferred_element_type=jnp.float32)
                    + b2a[...], 0.0)
    out_o[...] = (jnp.dot(t, w2b[...], preferred_element_type=jnp.float32)
                  + b2b[...])


def _tc_fin(*args):
    return pl.pallas_call(
        _fin_body,
        out_shape=jax.ShapeDtypeStruct((2 * _N, 128), jnp.float32),
    )(*args)


def kernel(emg_x, emg_edge_index, emg_edge_attr,
           eeg_x, eeg_edge_index, eeg_edge_attr, params):
    pe = params["emg_gat"]
    pg = params["eeg_gat"]
    gin = params["gin"]

    def pre(x, ea, p):
        av1 = jnp.stack([p["as1"], p["ad1"]], axis=1)
        return _tc_pre(x, p["W1"], av1, ea,
                       p["We1"], p["ae1"][None, :],
                       p["We2"], p["ae2"][None, :])

    h1_e, sd1_e, esc_e = pre(emg_x, emg_edge_attr, pe)
    h1_g, sd1_g, esc_g = pre(eeg_x, eeg_edge_attr, pg)

    src = jnp.stack([emg_edge_index[0], eeg_edge_index[0]])
    dst = jnp.stack([emg_edge_index[1], eeg_edge_index[1]])
    zeros = jnp.zeros((_N * _N,), jnp.float32)

    ssrc1 = jnp.stack([sd1_e[:, 0], sd1_g[:, 0]])
    sdst1 = jnp.stack([sd1_e[:, 1], sd1_g[:, 1]])
    esc1 = jnp.stack([esc_e[:, 0], esc_g[:, 0]])
    a1 = _sc_edge(ssrc1, sdst1, esc1, src, dst, zeros)
    a1 = a1.reshape(2, _N, _N)

    def mid(a, h1, p):
        av2 = jnp.stack([p["as2"], p["ad2"]], axis=1)
        return _tc_mid(a, h1, p["b1"][None, :], p["W2"], av2)

    h2_e, sd2_e = mid(a1[0], h1_e, pe)
    h2_g, sd2_g = mid(a1[1], h1_g, pg)

    ssrc2 = jnp.stack([sd2_e[:, 0], sd2_g[:, 0]])
    sdst2 = jnp.stack([sd2_e[:, 1], sd2_g[:, 1]])
    esc2 = jnp.stack([esc_e[:, 1], esc_g[:, 1]])
    a2 = _sc_edge(ssrc2, sdst2, esc2, src, dst, zeros)
    a2 = a2.reshape(2, _N, _N)

    prj_e = params["emg_proj"]
    prj_g = params["eeg_proj"]
    return _tc_fin(
        a2[0], h2_e, pe["b2"][None, :], prj_e["W"], prj_e["b"][None, :],
        a2[1], h2_g, pg["b2"][None, :], prj_g["W"], prj_g["b"][None, :],
        gin["W1a"], gin["b1a"][None, :], gin["W1b"], gin["b1b"][None, :],
        gin["W2a"], gin["b2a"][None, :], gin["W2b"], gin["b2b"][None, :])


# speedup vs baseline: 72.6618x; 1.3887x over previous
import functools

import jax
import jax.numpy as jnp
from jax import lax
from jax.experimental import pallas as pl
from jax.experimental.pallas import tpu as pltpu
from jax.experimental.pallas import tpu_sc as plsc

_N = 256
_E = 4096
_NC = 2
_NS = 16
_EPT = _E // _NS
_LANES = 16


def _pre_body(x_e, w1_e, as1_e, ad1_e, ea_e, we1_e, ae1_e, we2_e, ae2_e,
              x_g, w1_g, as1_g, ad1_g, ea_g, we1_g, ae1_g, we2_g, ae2_g,
              h1_o, ssrc_o, sdst_o, esc1_o, esc2_o):
    for b, (x, w1, a_s, a_d, ea, we1, ae1, we2, ae2) in enumerate((
            (x_e, w1_e, as1_e, ad1_e, ea_e, we1_e, ae1_e, we2_e, ae2_e),
            (x_g, w1_g, as1_g, ad1_g, ea_g, we1_g, ae1_g, we2_g, ae2_g))):
        h = jnp.dot(x[...], w1[...], preferred_element_type=jnp.float32)
        h1_o[b] = h
        ssrc_o[b] = jnp.sum(h * a_s[...], axis=1)
        sdst_o[b] = jnp.sum(h * a_d[...], axis=1)
        wc1 = jnp.sum(we1[...] * ae1[...], axis=1)
        wc2 = jnp.sum(we2[...] * ae2[...], axis=1)
        esc1_o[b] = jnp.sum(ea[...] * wc1[None, :], axis=1)
        esc2_o[b] = jnp.sum(ea[...] * wc2[None, :], axis=1)


def _tc_pre(*args):
    return pl.pallas_call(
        _pre_body,
        out_shape=[
            jax.ShapeDtypeStruct((2, _N, 512), jnp.float32),
            jax.ShapeDtypeStruct((2, _N), jnp.float32),
            jax.ShapeDtypeStruct((2, _N), jnp.float32),
            jax.ShapeDtypeStruct((2, _E), jnp.float32),
            jax.ShapeDtypeStruct((2, _E), jnp.float32),
        ],
    )(*args)


def _sc_edge(ssrc, sdst, esc, ei, zeros):
    mesh = plsc.VectorSubcoreMesh(
        core_axis_name="c", subcore_axis_name="s",
        num_cores=_NC, num_subcores=_NS)

    @functools.partial(
        pl.kernel,
        out_type=jax.ShapeDtypeStruct((2, _N * _N), jnp.float32),
        mesh=mesh,
        scratch_types=[
            pltpu.VMEM((_N,), jnp.float32),
            pltpu.VMEM((_N,), jnp.float32),
            pltpu.VMEM((_EPT,), jnp.float32),
            pltpu.VMEM((_EPT,), jnp.int32),
            pltpu.VMEM((_EPT,), jnp.int32),
            pltpu.VMEM((2, 128), jnp.float32),
            pltpu.VMEM((2, 128), jnp.int32),
            pltpu.VMEM_SHARED((_N * _N,), jnp.float32),
        ],
        compiler_params=pltpu.CompilerParams(needs_layout_passes=False),
    )
    def k(ssrc_h, sdst_h, esc_h, ei_h, zeros_h, a_out,
          ssrc_v, sdst_v, esc_v, src_v, dst_v, ex_v, idx_v, a_sh):
        c = lax.axis_index("c")
        s = lax.axis_index("s")
        base = s * _EPT
        pltpu.sync_copy(ssrc_h.at[c], ssrc_v)
        pltpu.sync_copy(sdst_h.at[c], sdst_v)
        pltpu.sync_copy(esc_h.at[c, pl.ds(base, _EPT)], esc_v)
        pltpu.sync_copy(ei_h.at[c, 0, pl.ds(base, _EPT)], src_v)
        pltpu.sync_copy(ei_h.at[c, 1, pl.ds(base, _EPT)], dst_v)

        @pl.when(s == 0)
        def _():
            pltpu.sync_copy(zeros_h, a_sh)

        for j in range(_EPT // _LANES):
            si = src_v[pl.ds(j * _LANES, _LANES)]
            di = dst_v[pl.ds(j * _LANES, _LANES)]
            sa = plsc.load_gather(ssrc_v, [si])
            sb = plsc.load_gather(sdst_v, [di])
            t = sa + sb + esc_v[pl.ds(j * _LANES, _LANES)]
            t = jnp.maximum(t, t * jnp.float32(0.2))
            ex_v[j // 8, pl.ds((j % 8) * _LANES, _LANES)] = jnp.exp(t)
            idx_v[j // 8, pl.ds((j % 8) * _LANES, _LANES)] = di * _N + si

        plsc.subcore_barrier()
        for g in range(2):
            pltpu.sync_copy(ex_v.at[g], a_sh.at[idx_v.at[g]], add=True)
        plsc.subcore_barrier()

        @pl.when(s == 0)
        def _():
            pltpu.sync_copy(a_sh, a_out.at[c])

    return k(ssrc, sdst, esc, ei, zeros)


def _mid_body(a, h1, b1_e, b1_g, w2_e, w2_g, as2_e, ad2_e, as2_g, ad2_g,
              h2_o, ssrc_o, sdst_o):
    for b, (b1, w2, a_s, a_d) in enumerate(((b1_e, w2_e, as2_e, ad2_e),
                                            (b1_g, w2_g, as2_g, ad2_g))):
        am = a[b]
        den = jnp.sum(am, axis=1, keepdims=True) + jnp.float32(1e-16)
        m = jnp.dot(am, h1[b], preferred_element_type=jnp.float32) / den
        t = jnp.maximum(m + b1[...], 0.0)
        h2 = jnp.dot(t, w2[...], preferred_element_type=jnp.float32)
        h2_o[b] = h2
        ssrc_o[b] = jnp.sum(h2 * a_s[...], axis=1)
        sdst_o[b] = jnp.sum(h2 * a_d[...], axis=1)


def _tc_mid(*args):
    return pl.pallas_call(
        _mid_body,
        out_shape=[
            jax.ShapeDtypeStruct((2, _N, 128), jnp.float32),
            jax.ShapeDtypeStruct((2, _N), jnp.float32),
            jax.ShapeDtypeStruct((2, _N), jnp.float32),
        ],
    )(*args)


def _fin_body(a, h2, b2_e, wp_e, bp_e, b2_g, wp_g, bp_g,
              w1a, b1a, w1b, b1b, w2a, b2a, w2b, b2b, out_o):
    feats = []
    for b, (b2, wp, bp) in enumerate(((b2_e, wp_e, bp_e),
                                      (b2_g, wp_g, bp_g))):
        am = a[b]
        den = jnp.sum(am, axis=1, keepdims=True) + jnp.float32(1e-16)
        o = jnp.dot(am, h2[b], preferred_element_type=jnp.float32) / den
        o = o + b2[...]
        feats.append(jnp.dot(o, wp[...], preferred_element_type=jnp.float32)
                     + bp[...])
    z = jnp.concatenate(feats, axis=0)
    t = z + jnp.sum(z, axis=0, keepdims=True)
    t = jnp.maximum(jnp.dot(t, w1a[...], preferred_element_type=jnp.float32)
                    + b1a[...], 0.0)
    t = jnp.dot(t, w1b[...], preferred_element_type=jnp.float32) + b1b[...]
    t = jnp.maximum(t, 0.0)
    t = t + jnp.sum(t, axis=0, keepdims=True)
    t = jnp.maximum(jnp.dot(t, w2a[...], preferred_element_type=jnp.float32)
                    + b2a[...], 0.0)
    out_o[...] = (jnp.dot(t, w2b[...], preferred_element_type=jnp.float32)
                  + b2b[...])


def _tc_fin(*args):
    return pl.pallas_call(
        _fin_body,
        out_shape=jax.ShapeDtypeStruct((2 * _N, 128), jnp.float32),
    )(*args)


def kernel(emg_x, emg_edge_index, emg_edge_attr,
           eeg_x, eeg_edge_index, eeg_edge_attr, params):
    pe = params["emg_gat"]
    pg = params["eeg_gat"]
    gin = params["gin"]
    row = lambda v: v[None, :]

    h1, ssrc1, sdst1, esc1, esc2 = _tc_pre(
        emg_x, pe["W1"], row(pe["as1"]), row(pe["ad1"]), emg_edge_attr,
        pe["We1"], row(pe["ae1"]), pe["We2"], row(pe["ae2"]),
        eeg_x, pg["W1"], row(pg["as1"]), row(pg["ad1"]), eeg_edge_attr,
        pg["We1"], row(pg["ae1"]), pg["We2"], row(pg["ae2"]))

    zeros = jnp.zeros((_N * _N,), jnp.float32)
    ei = jnp.stack([emg_edge_index, eeg_edge_index])
    a1 = _sc_edge(ssrc1, sdst1, esc1, ei, zeros)

    h2, ssrc2, sdst2 = _tc_mid(
        a1.reshape(2, _N, _N), h1, row(pe["b1"]), row(pg["b1"]),
        pe["W2"], pg["W2"], row(pe["as2"]), row(pe["ad2"]),
        row(pg["as2"]), row(pg["ad2"]))

    a2 = _sc_edge(ssrc2, sdst2, esc2, ei, zeros)

    prj_e = params["emg_proj"]
    prj_g = params["eeg_proj"]
    return _tc_fin(
        a2.reshape(2, _N, _N), h2,
        row(pe["b2"]), prj_e["W"], row(prj_e["b"]),
        row(pg["b2"]), prj_g["W"], row(prj_g["b"]),
        gin["W1a"], row(gin["b1a"]), gin["W1b"], row(gin["b1b"]),
        gin["W2a"], row(gin["b2a"]), gin["W2b"], row(gin["b2b"]))
